# Initial kernel scaffold; baseline (speedup 1.0000x reference)
import jax
import jax.numpy as jnp
from jax.experimental import pallas as pl


def kernel(t1s, t2s, t1_contexts, t2_contexts, table, att_mat, w_pred, b_pred):
    raise NotImplementedError("write your pallas kernel here")


# speedup vs baseline: 1.1043x; 1.1043x over previous
import functools

import jax
import jax.numpy as jnp
from jax import lax
from jax.experimental import pallas as pl
from jax.experimental.pallas import tpu as pltpu
from jax.experimental.pallas import tpu_sc as plsc

B, L, D = 16, 512, 128
NROW = 2 * B * L
CHUNK = 128


def _sc_gather(table, idx3, rows_per_w, n_chunks):
    info = plsc.get_sparse_core_info()
    nc, ns = info.num_cores, info.num_subcores
    nw = nc * ns
    mesh = plsc.VectorSubcoreMesh(core_axis_name="c", subcore_axis_name="s")

    @functools.partial(
        pl.kernel,
        mesh=mesh,
        out_type=jax.ShapeDtypeStruct((nw, rows_per_w, D), jnp.float32),
        scratch_types=[
            pltpu.VMEM((n_chunks, CHUNK), jnp.int32),
            pltpu.VMEM((rows_per_w, D), jnp.float32),
            pltpu.SemaphoreType.DMA,
        ],
    )
    def k(table_hbm, idx_hbm, out_hbm, idx_v, rows_v, sem):
        wid = lax.axis_index("s") * nc + lax.axis_index("c")
        pltpu.sync_copy(idx_hbm.at[wid], idx_v)
        copies = [
            pltpu.async_copy(
                table_hbm.at[idx_v.at[j]],
                rows_v.at[pl.ds(j * CHUNK, CHUNK)],
                sem,
            )
            for j in range(n_chunks)
        ]
        for c in copies:
            c.wait()
        pltpu.sync_copy(rows_v, out_hbm.at[wid])

    return k(table, idx3)


def _tc_body(e1_ref, e2_ref, att_ref, w_ref, logit_ref, sim_ref):
    e1 = e1_ref[0]
    e2 = e2_ref[0]
    ss1 = jnp.sum(e1 * e1)
    ss2 = jnp.sum(e2 * e2)
    inv = 1.0 / jnp.sqrt(ss1 * ss2)
    p = jnp.dot(e1, att_ref[...], preferred_element_type=jnp.float32)
    s_raw = lax.dot_general(
        p, e2, (((1,), (1,)), ((), ())), preferred_element_type=jnp.float32
    )
    s = jnp.tanh(s_raw * inv)
    sim_ref[0] = s

    rm = jnp.sum(s, axis=1, keepdims=True) * (1.0 / L)
    re = jnp.exp(rm - jnp.max(rm))
    rw = re / jnp.sum(re)
    na = lax.dot_general(
        rw, e1, (((0,), (0,)), ((), ())), preferred_element_type=jnp.float32
    )

    cm = jnp.sum(s, axis=0, keepdims=True) * (1.0 / L)
    ce = jnp.exp(cm - jnp.max(cm))
    cw = ce / jnp.sum(ce)
    nb = jnp.dot(cw, e2, preferred_element_type=jnp.float32)

    val = jnp.sum(na * nb * w_ref[...]) * inv
    logit_ref[...] = jnp.full((1, 1, D), val, dtype=jnp.float32)


def _tc_compute(g, att_mat, w_row):
    return pl.pallas_call(
        _tc_body,
        grid=(B,),
        in_specs=[
            pl.BlockSpec((1, L, D), lambda b: (b, 0, 0)),
            pl.BlockSpec((1, L, D), lambda b: (b + B, 0, 0)),
            pl.BlockSpec((D, D), lambda b: (0, 0)),
            pl.BlockSpec((1, D), lambda b: (0, 0)),
        ],
        out_specs=[
            pl.BlockSpec((1, 1, D), lambda b: (b, 0, 0)),
            pl.BlockSpec((1, L, L), lambda b: (b, 0, 0)),
        ],
        out_shape=[
            jax.ShapeDtypeStruct((B, 1, D), jnp.float32),
            jax.ShapeDtypeStruct((B, L, L), jnp.float32),
        ],
    )(g, g, att_mat, w_row)


def kernel(t1s, t2s, t1_contexts, t2_contexts, table, att_mat, w_pred, b_pred):
    idx = jnp.concatenate(
        [t1_contexts.reshape(-1), t2_contexts.reshape(-1)]
    ).astype(jnp.int32)
    info = plsc.get_sparse_core_info()
    nw = info.num_cores * info.num_subcores
    rows_per_w = NROW // nw
    n_chunks = rows_per_w // CHUNK
    idx3 = idx.reshape(nw, n_chunks, CHUNK)
    g = _sc_gather(table, idx3, rows_per_w, n_chunks).reshape(2 * B, L, D)
    logit3d, sim = _tc_compute(g, att_mat, w_pred.reshape(1, D))
    return logit3d[:, 0, 0] + b_pred[0], sim
